# baseline (speedup 1.0000x reference)
import jax
import jax.numpy as jnp
from jax.experimental import pallas as pl

_VOXEL_SIZE = jnp.array([0.4, 0.4, 6.4], dtype=jnp.float32)
_PC_MIN = jnp.array([-51.2, -51.2, -3.2], dtype=jnp.float32)
_PC_MAX = jnp.array([51.2, 51.2, 3.2], dtype=jnp.float32)
_GX, _GY = 256, 256
_C = 64
_NSEG = _GX * _GY + 1
_NPTS_BLK = 800


def _pfn_kernel(fin_ref, w_ref, b_ref, out_ref):
    fin = fin_ref[...]
    w = w_ref[...]
    acc = jnp.dot(fin, w, preferred_element_type=jnp.float32) + b_ref[...]
    out_ref[...] = jnp.maximum(acc, 0.0)


def _pfn(fin_pad, W_pad, b):
    n = fin_pad.shape[0]
    grid = (n // _NPTS_BLK,)
    return pl.pallas_call(
        _pfn_kernel,
        grid=grid,
        in_specs=[
            pl.BlockSpec((_NPTS_BLK, 16), lambda i: (i, 0)),
            pl.BlockSpec((16, _C), lambda i: (0, 0)),
            pl.BlockSpec((1, _C), lambda i: (0, 0)),
        ],
        out_specs=pl.BlockSpec((_NPTS_BLK, _C), lambda i: (i, 0)),
        out_shape=jax.ShapeDtypeStruct((n, _C), jnp.float32),
    )(fin_pad, W_pad, b.reshape(1, _C))


def _pillar_frame(pts, W, b):
    rel = (pts - _PC_MIN) / _VOXEL_SIZE
    coords = jnp.floor(rel).astype(jnp.int32)
    grid = jnp.array([_GX, _GY, 1], dtype=jnp.int32)
    valid = jnp.all((coords >= 0) & (coords < grid), axis=1)
    vid = coords[:, 0] * _GY + coords[:, 1]
    vid = jnp.where(valid, vid, _GX * _GY)
    vmask = valid.astype(jnp.float32)[:, None]
    counts = jax.ops.segment_sum(vmask[:, 0], vid, num_segments=_NSEG)
    sums = jax.ops.segment_sum(pts * vmask, vid, num_segments=_NSEG)
    mean = sums / jnp.clip(counts, 1.0)[:, None]
    cluster_off = pts - mean[vid]
    center = (coords.astype(jnp.float32) + 0.5) * _VOXEL_SIZE + _PC_MIN
    center_off = pts - center
    fin = jnp.concatenate([pts, cluster_off, center_off], axis=1)
    n = pts.shape[0]
    fin_pad = jnp.concatenate([fin, jnp.zeros((n, 7), jnp.float32)], axis=1)
    W_pad = jnp.concatenate([W, jnp.zeros((7, _C), jnp.float32)], axis=0)
    point_feats = _pfn(fin_pad, W_pad, b) * vmask
    pooled = jax.ops.segment_sum(point_feats, vid, num_segments=_NSEG)
    pooled = pooled / jnp.clip(counts, 1.0)[:, None]
    pooled = pooled[: _GX * _GY]
    canvas = pooled.reshape(_GX, _GY, _C).transpose(2, 0, 1)
    return canvas, point_feats, pooled


def kernel(pc0s, pc1s, training_flag, W, b):
    f = lambda p: _pillar_frame(p, W, b)
    canvas0, pf0, pooled0 = jax.vmap(f)(pc0s)
    canvas1, _, _ = jax.vmap(f)(pc1s)
    tensor_4d = jnp.stack([canvas0, canvas1], axis=-1)
    return tensor_4d, pf0, pooled0


# speedup vs baseline: 1.1598x; 1.1598x over previous
import jax
import jax.numpy as jnp
from jax import lax
from jax.experimental import pallas as pl

_GX, _GY, _C = 256, 256, 64
_N = 100000
_TROW = 56
_NPAD = 16 * _TROW * 128
_NCLOUD = 4
_DUMP = _GX * _GY
_ACC_R = 66048
_STRIPE = _ACC_R // 16

_VOX4 = (0.4, 0.4, 6.4, 1.0)
_MIN4 = (-51.2, -51.2, -3.2, -1.0)
_GRID4 = (256.0, 256.0, 1.0, 1e30)
_CNT_SEL = (0.0, 0.0, 0.0, 1.0, 0.0)

_BPT = 2048


def _lane_const(shape, values):
    l = lax.broadcasted_iota(jnp.int32, shape, len(shape) - 1)
    out = jnp.full(shape, values[-1], jnp.float32)
    for i in range(len(values) - 2, -1, -1):
        out = jnp.where(l == i, jnp.float32(values[i]), out)
    return out


def _consts(shape):
    vox = _lane_const(shape, _VOX4)
    mn = _lane_const(shape, _MIN4)
    gr = _lane_const(shape, _GRID4)
    return vox, mn, gr


def _vid_kernel(pc_ref, vid_ref):
    p = pc_ref[0]
    vox, mn, gr = _consts(p.shape)
    rel = (p - mn) / vox
    cf = jnp.floor(rel)
    valid = jnp.all((rel >= 0.0) & (cf < gr), axis=1)
    vidf = jnp.sum(cf * _lane_const(p.shape, (float(_GY), 1.0, 0.0, 0.0)),
                   axis=1)
    vidf = jnp.where(valid, vidf, float(_DUMP))
    vid_ref[0, 0] = vidf.astype(jnp.int32).reshape(8, 128)


def _stage_a(pc4):
    return pl.pallas_call(
        _vid_kernel,
        grid=(_NCLOUD, _NPAD // 1024),
        in_specs=[pl.BlockSpec((1, 1024, 4), lambda c, i: (c, i, 0))],
        out_specs=pl.BlockSpec((1, 1, 8, 128),
                               lambda c, i: (c, i // 7, i % 7, 0)),
        out_shape=jax.ShapeDtypeStruct((_NCLOUD, 16, _TROW, 128), jnp.int32),
    )(pc4)


def _pfn_kernel(pc_ref, mr_ref, ws_ref, w1_ref, w2_ref, b_ref, out_ref):
    p = pc_ref[0]
    mr = mr_ref[0]
    vox, mn, gr = _consts(p.shape)
    rel = (p - mn) / vox
    cf = jnp.floor(rel)
    valid = jnp.all((rel >= 0.0) & (cf < gr), axis=1, keepdims=True)
    cnt = jnp.sum(mr * _lane_const(mr.shape, _CNT_SEL),
                  axis=1, keepdims=True)
    mean = mr / jnp.maximum(cnt, 1.0)
    center = (cf + 0.5) * vox + mn
    acc = (jnp.dot(p, ws_ref[...], preferred_element_type=jnp.float32)
           - jnp.dot(mean, w1_ref[...], preferred_element_type=jnp.float32)
           - jnp.dot(center, w2_ref[...], preferred_element_type=jnp.float32)
           + b_ref[...])
    out_ref[0] = jnp.maximum(acc, 0.0) * valid


def _stage_d(pc4, meanrows, ws, w1p, w2p, b):
    return pl.pallas_call(
        _pfn_kernel,
        grid=(_NCLOUD, _NPAD // _BPT),
        in_specs=[
            pl.BlockSpec((1, _BPT, 4), lambda c, i: (c, i, 0)),
            pl.BlockSpec((1, _BPT, 16), lambda c, i: (c, i, 0)),
            pl.BlockSpec((4, _C), lambda c, i: (0, 0)),
            pl.BlockSpec((16, _C), lambda c, i: (0, 0)),
            pl.BlockSpec((4, _C), lambda c, i: (0, 0)),
            pl.BlockSpec((1, _C), lambda c, i: (0, 0)),
        ],
        out_specs=pl.BlockSpec((1, _BPT, _C), lambda c, i: (c, i, 0)),
        out_shape=jax.ShapeDtypeStruct((_NCLOUD, _NPAD, _C), jnp.float32),
    )(pc4, meanrows, ws, w1p, w2p, b.reshape(1, _C))


def _canvas_kernel(ps_ref, sums_ref, out_ref):
    ps = ps_ref[0]
    s16 = sums_ref[0]
    cnt = jnp.sum(s16 * _lane_const(s16.shape, _CNT_SEL), axis=1,
                  keepdims=True)
    pooled = ps / jnp.maximum(cnt, 1.0)
    out_ref[0, :, 0] = pooled.T.reshape(_C, 8, 256)


def _stage_f1(psum, sums):
    return pl.pallas_call(
        _canvas_kernel,
        grid=(_NCLOUD, 32),
        in_specs=[
            pl.BlockSpec((1, 2048, _C), lambda c, x: (c, x, 0)),
            pl.BlockSpec((1, 2048, 16), lambda c, x: (c, x, 0)),
        ],
        out_specs=pl.BlockSpec(
            (1, _C, 1, 8, 256),
            lambda c, x: (c % 2, 0, c // 2, x, 0)),
        out_shape=jax.ShapeDtypeStruct((2, _C, 2, _GX, _GY), jnp.float32),
    )(psum, sums)


def _pooled_kernel(ps_ref, sums_ref, out_ref):
    ps = ps_ref[0]
    s16 = sums_ref[0]
    cnt = jnp.sum(s16 * _lane_const(s16.shape, _CNT_SEL), axis=1,
                  keepdims=True)
    out_ref[0] = ps / jnp.maximum(cnt, 1.0)


def _stage_f2(psum, sums):
    return pl.pallas_call(
        _pooled_kernel,
        grid=(2, 32),
        in_specs=[
            pl.BlockSpec((1, 2048, _C), lambda c, x: (c, x, 0)),
            pl.BlockSpec((1, 2048, 16), lambda c, x: (c, x, 0)),
        ],
        out_specs=pl.BlockSpec((1, 2048, _C), lambda c, x: (c, x, 0)),
        out_shape=jax.ShapeDtypeStruct((2, _GX * _GY, _C), jnp.float32),
    )(psum, sums)


def kernel(pc0s, pc1s, training_flag, W, b):
    pcs = jnp.concatenate([pc0s, pc1s], axis=0)
    pcs = jnp.pad(pcs, ((0, 0), (0, _NPAD - _N), (0, 0)),
                  constant_values=1e9)
    pc4 = jnp.pad(pcs, ((0, 0), (0, 0), (0, 1)))
    nc = pcs.shape[0]
    pts16 = jnp.concatenate(
        [pcs, jnp.ones((nc, _NPAD, 1), jnp.float32),
         jnp.zeros((nc, _NPAD, 12), jnp.float32)], axis=-1)

    w0, w1, w2 = W[0:3], W[3:6], W[6:9]
    zrow = jnp.zeros((1, _C), jnp.float32)
    ws = jnp.concatenate([w0 + w1 + w2, zrow], axis=0)
    w1p = jnp.concatenate([w1, jnp.zeros((13, _C), jnp.float32)], axis=0)
    w2p = jnp.concatenate([w2, zrow], axis=0)

    vid = _stage_a(pc4)
    vidf = vid.reshape(_NCLOUD, _NPAD)
    sums = jax.vmap(
        lambda x, v: jax.ops.segment_sum(x, v, num_segments=_ACC_R)
    )(pts16, vidf)
    meanrows = jax.vmap(lambda s, v: s[v])(sums, vidf)
    feats = _stage_d(pc4, meanrows, ws, w1p, w2p, b)
    psum = jax.vmap(
        lambda x, v: jax.ops.segment_sum(x, v, num_segments=_ACC_R)
    )(feats, vidf)
    tensor_4d = jnp.moveaxis(_stage_f1(psum, sums), 2, -1)
    pooled0 = _stage_f2(psum[0:2], sums[0:2])
    pf0 = feats[0:2, :_N, :]
    return tensor_4d, pf0, pooled0


# speedup vs baseline: 1.4532x; 1.2529x over previous
import jax
import jax.numpy as jnp
from jax import lax
from jax.experimental import pallas as pl

_GX, _GY, _C = 256, 256, 64
_N = 100000
_NPAD = 102400
_NCLOUD = 4
_DUMP = _GX * _GY
_ACC_R = 66048
_STRIPE = _ACC_R // 16

_VOX4 = (0.4, 0.4, 6.4, 1.0)
_MIN4 = (-51.2, -51.2, -3.2, -1.0)
_GRID4 = (256.0, 256.0, 1.0, 1e30)
_CNT_SEL = (0.0, 0.0, 0.0, 1.0)

_BPT = 2048


def _lane_const(shape, values):
    l = lax.broadcasted_iota(jnp.int32, shape, len(shape) - 1)
    out = jnp.full(shape, values[-1], jnp.float32)
    for i in range(len(values) - 2, -1, -1):
        out = jnp.where(l == i, jnp.float32(values[i]), out)
    return out


def _consts(shape):
    vox = _lane_const(shape, _VOX4)
    mn = _lane_const(shape, _MIN4)
    gr = _lane_const(shape, _GRID4)
    return vox, mn, gr


def _vid_kernel(pc_ref, vid_ref):
    p = pc_ref[0]
    vox, mn, gr = _consts(p.shape)
    rel = (p - mn) / vox
    cf = jnp.floor(rel)
    valid = jnp.all((rel >= 0.0) & (cf < gr), axis=1)
    vidf = jnp.sum(cf * _lane_const(p.shape, (float(_GY), 1.0, 0.0, 0.0)),
                   axis=1)
    vidf = jnp.where(valid, vidf, float(_DUMP))
    vid_ref[0] = vidf.astype(jnp.int32).reshape(8, 128)


def _stage_a(pc4):
    return pl.pallas_call(
        _vid_kernel,
        grid=(_NCLOUD, _NPAD // 1024),
        in_specs=[pl.BlockSpec((1, 1024, 4), lambda c, i: (c, i, 0))],
        out_specs=pl.BlockSpec((1, 8, 128), lambda c, i: (c, i, 0)),
        out_shape=jax.ShapeDtypeStruct((_NCLOUD, _NPAD // 128, 128),
                                       jnp.int32),
    )(pc4)


def _pfn_kernel(pc_ref, mr_ref, ws_ref, w1_ref, w2_ref, b_ref, out_ref):
    p = pc_ref[0]
    mr = mr_ref[0]
    vox, mn, gr = _consts(p.shape)
    rel = (p - mn) / vox
    cf = jnp.floor(rel)
    valid = jnp.all((rel >= 0.0) & (cf < gr), axis=1, keepdims=True)
    cnt = jnp.sum(mr * _lane_const(mr.shape, _CNT_SEL),
                  axis=1, keepdims=True)
    mean = mr / jnp.maximum(cnt, 1.0)
    center = (cf + 0.5) * vox + mn
    acc = (jnp.dot(p, ws_ref[...], preferred_element_type=jnp.float32)
           - jnp.dot(mean, w1_ref[...], preferred_element_type=jnp.float32)
           - jnp.dot(center, w2_ref[...], preferred_element_type=jnp.float32)
           + b_ref[...])
    out_ref[0] = jnp.maximum(acc, 0.0) * valid


def _stage_d(pc4, meanrows, ws, w1p, w2p, b):
    return pl.pallas_call(
        _pfn_kernel,
        grid=(_NCLOUD, _NPAD // _BPT),
        in_specs=[
            pl.BlockSpec((1, _BPT, 4), lambda c, i: (c, i, 0)),
            pl.BlockSpec((1, _BPT, 4), lambda c, i: (c, i, 0)),
            pl.BlockSpec((4, _C), lambda c, i: (0, 0)),
            pl.BlockSpec((4, _C), lambda c, i: (0, 0)),
            pl.BlockSpec((4, _C), lambda c, i: (0, 0)),
            pl.BlockSpec((1, _C), lambda c, i: (0, 0)),
        ],
        out_specs=pl.BlockSpec((1, _BPT, _C), lambda c, i: (c, i, 0)),
        out_shape=jax.ShapeDtypeStruct((_NCLOUD, _NPAD, _C), jnp.float32),
    )(pc4, meanrows, ws, w1p, w2p, b.reshape(1, _C))


def _canvas_kernel(ps_ref, sums_ref, out_ref):
    ps = ps_ref[0]
    s4 = sums_ref[0]
    cnt = jnp.sum(s4 * _lane_const(s4.shape, _CNT_SEL), axis=1,
                  keepdims=True)
    pooled = ps / jnp.maximum(cnt, 1.0)
    out_ref[0, :, 0] = pooled.T.reshape(_C, 8, 256)


def _stage_f1(psum, sums):
    return pl.pallas_call(
        _canvas_kernel,
        grid=(_NCLOUD, 32),
        in_specs=[
            pl.BlockSpec((1, 2048, _C), lambda c, x: (c, x, 0)),
            pl.BlockSpec((1, 2048, 4), lambda c, x: (c, x, 0)),
        ],
        out_specs=pl.BlockSpec(
            (1, _C, 1, 8, 256),
            lambda c, x: (c % 2, 0, c // 2, x, 0)),
        out_shape=jax.ShapeDtypeStruct((2, _C, 2, _GX, _GY), jnp.float32),
    )(psum, sums)


def _pooled_kernel(ps_ref, sums_ref, out_ref):
    ps = ps_ref[0]
    s4 = sums_ref[0]
    cnt = jnp.sum(s4 * _lane_const(s4.shape, _CNT_SEL), axis=1,
                  keepdims=True)
    out_ref[0] = ps / jnp.maximum(cnt, 1.0)


def _stage_f2(psum, sums):
    return pl.pallas_call(
        _pooled_kernel,
        grid=(2, 32),
        in_specs=[
            pl.BlockSpec((1, 2048, _C), lambda c, x: (c, x, 0)),
            pl.BlockSpec((1, 2048, 4), lambda c, x: (c, x, 0)),
        ],
        out_specs=pl.BlockSpec((1, 2048, _C), lambda c, x: (c, x, 0)),
        out_shape=jax.ShapeDtypeStruct((2, _GX * _GY, _C), jnp.float32),
    )(psum, sums)


def kernel(pc0s, pc1s, training_flag, W, b):
    pcs = jnp.concatenate([pc0s, pc1s], axis=0)
    pcs = jnp.pad(pcs, ((0, 0), (0, _NPAD - _N), (0, 0)),
                  constant_values=1e9)
    pc4 = jnp.pad(pcs, ((0, 0), (0, 0), (0, 1)))
    nc = pcs.shape[0]
    pts4 = jnp.concatenate(
        [pcs, jnp.ones((nc, _NPAD, 1), jnp.float32)], axis=-1)

    w0, w1, w2 = W[0:3], W[3:6], W[6:9]
    zrow = jnp.zeros((1, _C), jnp.float32)
    ws = jnp.concatenate([w0 + w1 + w2, zrow], axis=0)
    w1p = jnp.concatenate([w1, zrow], axis=0)
    w2p = jnp.concatenate([w2, zrow], axis=0)

    vid = _stage_a(pc4)
    vidf = vid.reshape(_NCLOUD, _NPAD)
    sums = jax.vmap(
        lambda x, v: jax.ops.segment_sum(x, v, num_segments=_ACC_R)
    )(pts4, vidf)
    meanrows = jax.vmap(lambda s, v: s[v])(sums, vidf)
    feats = _stage_d(pc4, meanrows, ws, w1p, w2p, b)
    psum = jax.vmap(
        lambda x, v: jax.ops.segment_sum(x, v, num_segments=_ACC_R)
    )(feats, vidf)
    tensor_4d = jnp.moveaxis(_stage_f1(psum, sums), 2, -1)
    pooled0 = _stage_f2(psum[0:2], sums[0:2])
    pf0 = feats[0:2, :_N, :]
    return tensor_4d, pf0, pooled0
